# Initial kernel scaffold; baseline (speedup 1.0000x reference)
import jax
import jax.numpy as jnp
from jax.experimental import pallas as pl


def kernel(x, edge_index, a_values, W, b, gamma, beta):
    raise NotImplementedError("write your pallas kernel here")


# speedup vs baseline: 2.5027x; 2.5027x over previous
import functools

import jax
import jax.numpy as jnp
from jax import lax
from jax.experimental import pallas as pl
from jax.experimental.pallas import tpu as pltpu
from jax.experimental.pallas import tpu_sc as plsc

N_NODES = 10000
D = 128
DH = D // 2
N_PAD = 10240
E_PAD = 327680
EW = E_PAD // 16
CHUNK = 512
NCHUNK = EW // CHUNK
ZROWS = 128
TROWS = N_PAD // 16


def _build_sc_agg():
    mesh = plsc.VectorSubcoreMesh(core_axis_name="c", subcore_axis_name="s")

    @functools.partial(
        pl.kernel,
        mesh=mesh,
        compiler_params=pltpu.CompilerParams(use_tc_tiling_on_sc=False),
        out_type=jax.ShapeDtypeStruct((2, N_PAD, DH), jnp.float32),
        scratch_types=[
            pltpu.VMEM((CHUNK // 128, 128), jnp.int32),
            pltpu.VMEM((CHUNK // 128, 128), jnp.int32),
            pltpu.VMEM((CHUNK,), jnp.float32),
            pltpu.VMEM((CHUNK, DH), jnp.float32),
            pltpu.VMEM_SHARED((N_PAD, DH), jnp.float32),
            pltpu.SemaphoreType.DMA,
        ],
    )
    def sc_agg(xh_hbm, col_hbm, row_hbm, a_hbm, out_hbm,
               col_v, row_v, a_v, rows_v, agg_sh, sem):
        cid = lax.axis_index("c")
        sid = lax.axis_index("s")
        zero16 = jnp.zeros((16,), jnp.float32)

        def zrow(i, carry):
            for j in range(DH // 16):
                rows_v[i, pl.ds(j * 16, 16)] = zero16
            return carry

        lax.fori_loop(0, ZROWS, zrow, 0)
        tile_row0 = sid * TROWS
        for k in range(TROWS // ZROWS):
            pltpu.sync_copy(rows_v.at[pl.ds(0, ZROWS)],
                            agg_sh.at[pl.ds(tile_row0 + k * ZROWS, ZROWS)])
        plsc.subcore_barrier()

        _dnums = lax.GatherDimensionNumbers(
            offset_dims=(), collapsed_slice_dims=(0,), start_index_map=(0,))

        def chunk_body(c, carry):
            rbase = sid * (EW // 128) + c * (CHUNK // 128)
            ebase = sid * EW + c * CHUNK
            pltpu.sync_copy(col_hbm.at[cid, pl.ds(rbase, CHUNK // 128)], col_v)
            pltpu.sync_copy(row_hbm.at[pl.ds(rbase, CHUNK // 128)], row_v)
            pltpu.sync_copy(a_hbm.at[pl.ds(ebase, CHUNK)], a_v)
            cps = [
                pltpu.async_copy(xh_hbm.at[col_v.at[j]],
                                 rows_v.at[pl.ds(j * 128, 128)], sem)
                for j in range(CHUNK // 128)
            ]
            for cp in cps:
                cp.wait()

            def grp(g, carry2):
                e0 = g * 16
                a16 = a_v[pl.ds(e0, 16)]
                for l in range(16):
                    e = e0 + l
                    bc = lax.gather(
                        a16, jnp.full((16, 1), l, jnp.int32), _dnums, (1,),
                        mode=lax.GatherScatterMode.PROMISE_IN_BOUNDS)
                    for j in range(DH // 16):
                        rows_v[e, pl.ds(j * 16, 16)] = (
                            rows_v[e, pl.ds(j * 16, 16)] * bc)
                return carry2

            lax.fori_loop(0, CHUNK // 16, grp, 0)

            for j in range(CHUNK // 128):
                pltpu.sync_copy(rows_v.at[pl.ds(j * 128, 128)],
                                agg_sh.at[row_v.at[j]], add=True)
            return carry

        lax.fori_loop(0, NCHUNK, chunk_body, 0)

        plsc.subcore_barrier()
        for k in range(TROWS // ZROWS):
            r0 = tile_row0 + k * ZROWS
            pltpu.sync_copy(agg_sh.at[pl.ds(r0, ZROWS)],
                            rows_v.at[pl.ds(0, ZROWS)])
            pltpu.sync_copy(rows_v.at[pl.ds(0, ZROWS)],
                            out_hbm.at[cid, pl.ds(r0, ZROWS)])

    return sc_agg


_sc_agg = _build_sc_agg()

_MB = 1000


def _tc_body(agg_ref, w_ref, b_ref, g_ref, bt_ref, o_ref):
    acc = jnp.concatenate([agg_ref[0], agg_ref[1]], axis=-1)
    h = jnp.dot(acc, w_ref[...], preferred_element_type=jnp.float32)
    h = jnp.maximum(h + b_ref[...], 0.0)
    o_ref[...] = (g_ref[...] * h) / jnp.sqrt(jnp.float32(1.0 + 1e-3)) + bt_ref[...]


def _tc_finish(partials, W, b, gamma, beta):
    return pl.pallas_call(
        _tc_body,
        grid=(N_NODES // _MB,),
        in_specs=[
            pl.BlockSpec((2, _MB, DH), lambda i: (0, i, 0)),
            pl.BlockSpec((D, D), lambda i: (0, 0)),
            pl.BlockSpec((1, D), lambda i: (0, 0)),
            pl.BlockSpec((1, D), lambda i: (0, 0)),
            pl.BlockSpec((1, D), lambda i: (0, 0)),
        ],
        out_specs=pl.BlockSpec((_MB, D), lambda i: (i, 0)),
        out_shape=jax.ShapeDtypeStruct((N_NODES, D), jnp.float32),
    )(partials, W, b.reshape(1, D), gamma.reshape(1, D), beta.reshape(1, D))


def kernel(x, edge_index, a_values, W, b, gamma, beta):
    row = edge_index[0].astype(jnp.int32)
    col = edge_index[1].astype(jnp.int32)
    n_edges = row.shape[0]
    pad = E_PAD - n_edges
    ipad = jnp.zeros((pad,), jnp.int32)
    rowp = jnp.concatenate([row, ipad]).reshape(E_PAD // 128, 128)
    colp = jnp.concatenate([col, ipad]).reshape(E_PAD // 128, 128)
    colboth = jnp.stack([colp, colp + N_NODES])
    ap = jnp.concatenate([a_values, jnp.zeros((pad,), jnp.float32)])
    xh = jnp.concatenate([x[:, :DH], x[:, DH:]], axis=0)
    partials = _sc_agg(xh, colboth, rowp, ap)
    return _tc_finish(partials, W, b, gamma, beta)


# speedup vs baseline: 3.6518x; 1.4591x over previous
import functools

import jax
import jax.numpy as jnp
from jax import lax
from jax.experimental import pallas as pl
from jax.experimental.pallas import tpu as pltpu
from jax.experimental.pallas import tpu_sc as plsc

N_NODES = 10000
D = 128
DH = D // 2
N_PAD = 10240
E_PAD = 327680
EW = E_PAD // 16
ERW = EW // 128
CHUNK = 512
CR = CHUNK // 128
NCHUNK = EW // CHUNK
NRING = 4
ZROWS = 128
TROWS = N_PAD // 16


def _build_sc_agg():
    mesh = plsc.VectorSubcoreMesh(core_axis_name="c", subcore_axis_name="s")

    @functools.partial(
        pl.kernel,
        mesh=mesh,
        compiler_params=pltpu.CompilerParams(use_tc_tiling_on_sc=False),
        out_type=jax.ShapeDtypeStruct((2, N_PAD, DH), jnp.float32),
        scratch_types=[
            pltpu.VMEM((NRING, CR, 128), jnp.int32),
            pltpu.VMEM((NRING, CR, 128), jnp.int32),
            pltpu.VMEM((NRING, CHUNK), jnp.float32),
            pltpu.VMEM((CHUNK, DH), jnp.float32),
            pltpu.VMEM((CHUNK, DH), jnp.float32),
            pltpu.VMEM_SHARED((N_PAD, DH), jnp.float32),
            pltpu.SemaphoreType.DMA,
            pltpu.SemaphoreType.DMA,
            pltpu.SemaphoreType.DMA,
        ],
    )
    def sc_agg(xh_hbm, col_hbm, row_hbm, a_hbm, out_hbm,
               col_v, row_v, a_v, rows0_v, rows1_v, agg_sh,
               sem_i, sem_g, sem_s):
        cid = lax.axis_index("c")
        sid = lax.axis_index("s")
        zero16 = jnp.zeros((16,), jnp.float32)
        rows_bufs = (rows0_v, rows1_v)

        def issue_idx(c):
            slot = lax.rem(c, NRING)
            rbase = sid * ERW + c * CR
            pltpu.async_copy(col_hbm.at[cid, pl.ds(rbase, CR)],
                             col_v.at[slot], sem_i)
            pltpu.async_copy(row_hbm.at[pl.ds(rbase, CR)],
                             row_v.at[slot], sem_i)
            pltpu.async_copy(a_hbm.at[pl.ds(sid * EW + c * CHUNK, CHUNK)],
                             a_v.at[slot], sem_i)

        def wait_idx(c):
            slot = lax.rem(c, NRING)
            rbase = sid * ERW + c * CR
            pltpu.make_async_copy(col_hbm.at[cid, pl.ds(rbase, CR)],
                                  col_v.at[slot], sem_i).wait()
            pltpu.make_async_copy(row_hbm.at[pl.ds(rbase, CR)],
                                  row_v.at[slot], sem_i).wait()
            pltpu.make_async_copy(a_hbm.at[pl.ds(sid * EW + c * CHUNK, CHUNK)],
                                  a_v.at[slot], sem_i).wait()

        def issue_gathers(c, buf):
            slot = lax.rem(c, NRING)
            for j in range(CR):
                pltpu.async_copy(xh_hbm.at[col_v.at[slot, j]],
                                 buf.at[pl.ds(j * 128, 128)], sem_g)

        def wait_gathers(c, buf):
            slot = lax.rem(c, NRING)
            for j in range(CR):
                pltpu.make_async_copy(xh_hbm.at[col_v.at[slot, j]],
                                      buf.at[pl.ds(j * 128, 128)],
                                      sem_g).wait()

        def issue_scatters(c, buf):
            slot = lax.rem(c, NRING)
            for j in range(CR):
                pltpu.async_copy(buf.at[pl.ds(j * 128, 128)],
                                 agg_sh.at[row_v.at[slot, j]],
                                 sem_s, add=True)

        def wait_scatters(c, buf):
            slot = lax.rem(c, NRING)
            for j in range(CR):
                pltpu.make_async_copy(buf.at[pl.ds(j * 128, 128)],
                                      agg_sh.at[row_v.at[slot, j]],
                                      sem_s).wait()

        issue_idx(0)
        issue_idx(1)

        def zrow(i, carry):
            for j in range(DH // 16):
                rows0_v[i, pl.ds(j * 16, 16)] = zero16
            return carry

        lax.fori_loop(0, ZROWS, zrow, 0)
        tile_row0 = sid * TROWS
        for k in range(TROWS // ZROWS):
            pltpu.sync_copy(rows0_v.at[pl.ds(0, ZROWS)],
                            agg_sh.at[pl.ds(tile_row0 + k * ZROWS, ZROWS)])
        wait_idx(0)
        issue_gathers(0, rows0_v)
        plsc.subcore_barrier()

        _dnums = lax.GatherDimensionNumbers(
            offset_dims=(), collapsed_slice_dims=(0,), start_index_map=(0,))

        def scale(c, buf):
            slot = lax.rem(c, NRING)

            def grp(g, carry2):
                a16 = a_v[slot, pl.ds(g * 16, 16)]
                for l in range(16):
                    e = g * 16 + l
                    bc = lax.gather(
                        a16, jnp.full((16, 1), l, jnp.int32), _dnums, (1,),
                        mode=lax.GatherScatterMode.PROMISE_IN_BOUNDS)
                    for j in range(DH // 16):
                        buf[e, pl.ds(j * 16, 16)] = (
                            buf[e, pl.ds(j * 16, 16)] * bc)
                return carry2

            lax.fori_loop(0, CHUNK // 16, grp, 0)

        def body2(c2, carry):
            for b in range(2):
                c = c2 * 2 + b
                buf = rows_bufs[b]
                nbuf = rows_bufs[1 - b]
                wait_gathers(c, buf)

                @pl.when(c >= 1)
                def _():
                    wait_scatters(c - 1, nbuf)

                @pl.when(c + 1 < NCHUNK)
                def _():
                    wait_idx(c + 1)
                    issue_gathers(c + 1, nbuf)

                @pl.when(c + 2 < NCHUNK)
                def _():
                    issue_idx(c + 2)

                scale(c, buf)
                issue_scatters(c, buf)
            return carry

        lax.fori_loop(0, NCHUNK // 2, body2, 0)
        wait_scatters(NCHUNK - 1, rows_bufs[(NCHUNK - 1) % 2])

        plsc.subcore_barrier()
        for k in range(TROWS // ZROWS):
            r0 = tile_row0 + k * ZROWS
            pltpu.sync_copy(agg_sh.at[pl.ds(r0, ZROWS)],
                            rows0_v.at[pl.ds(0, ZROWS)])
            pltpu.sync_copy(rows0_v.at[pl.ds(0, ZROWS)],
                            out_hbm.at[cid, pl.ds(r0, ZROWS)])

    return sc_agg


_sc_agg = _build_sc_agg()

_MB = 1000


def _tc_body(agg_ref, w_ref, b_ref, g_ref, bt_ref, o_ref):
    acc = jnp.concatenate([agg_ref[0], agg_ref[1]], axis=-1)
    h = jnp.dot(acc, w_ref[...], preferred_element_type=jnp.float32)
    h = jnp.maximum(h + b_ref[...], 0.0)
    o_ref[...] = (g_ref[...] * h) / jnp.sqrt(jnp.float32(1.0 + 1e-3)) + bt_ref[...]


def _tc_finish(partials, W, b, gamma, beta):
    return pl.pallas_call(
        _tc_body,
        grid=(N_NODES // _MB,),
        in_specs=[
            pl.BlockSpec((2, _MB, DH), lambda i: (0, i, 0)),
            pl.BlockSpec((D, D), lambda i: (0, 0)),
            pl.BlockSpec((1, D), lambda i: (0, 0)),
            pl.BlockSpec((1, D), lambda i: (0, 0)),
            pl.BlockSpec((1, D), lambda i: (0, 0)),
        ],
        out_specs=pl.BlockSpec((_MB, D), lambda i: (i, 0)),
        out_shape=jax.ShapeDtypeStruct((N_NODES, D), jnp.float32),
    )(partials, W, b.reshape(1, D), gamma.reshape(1, D), beta.reshape(1, D))


def kernel(x, edge_index, a_values, W, b, gamma, beta):
    row = edge_index[0].astype(jnp.int32)
    col = edge_index[1].astype(jnp.int32)
    n_edges = row.shape[0]
    pad = E_PAD - n_edges
    ipad = jnp.zeros((pad,), jnp.int32)
    rowp = jnp.concatenate([row, ipad]).reshape(E_PAD // 128, 128)
    colp = jnp.concatenate([col, ipad]).reshape(E_PAD // 128, 128)
    colboth = jnp.stack([colp, colp + N_NODES])
    ap = jnp.concatenate([a_values, jnp.zeros((pad,), jnp.float32)])
    xh = jnp.concatenate([x[:, :DH], x[:, DH:]], axis=0)
    partials = _sc_agg(xh, colboth, rowp, ap)
    return _tc_finish(partials, W, b, gamma, beta)


# speedup vs baseline: 5.0519x; 1.3834x over previous
import functools

import jax
import jax.numpy as jnp
from jax import lax
from jax.experimental import pallas as pl
from jax.experimental.pallas import tpu as pltpu
from jax.experimental.pallas import tpu_sc as plsc

N_NODES = 10000
D = 128
DH = D // 2
N_PAD = 10240
E_PAD = 327680
EW = E_PAD // 16
ERW = EW // 128
CHUNK = 256
CR = CHUNK // 128
NCHUNK = EW // CHUNK
NRING = 6
NBUFS = 4
ZROWS = 128
TROWS = N_PAD // 16


def _build_sc_agg():
    mesh = plsc.VectorSubcoreMesh(core_axis_name="c", subcore_axis_name="s")

    @functools.partial(
        pl.kernel,
        mesh=mesh,
        compiler_params=pltpu.CompilerParams(use_tc_tiling_on_sc=False),
        out_type=jax.ShapeDtypeStruct((2, N_PAD, DH), jnp.float32),
        scratch_types=[
            pltpu.VMEM((NRING, CR, 128), jnp.int32),
            pltpu.VMEM((NRING, CR, 128), jnp.int32),
            pltpu.VMEM((NRING, CHUNK), jnp.float32),
            pltpu.VMEM((CHUNK, DH), jnp.float32),
            pltpu.VMEM((CHUNK, DH), jnp.float32),
            pltpu.VMEM((CHUNK, DH), jnp.float32),
            pltpu.VMEM((CHUNK, DH), jnp.float32),
            pltpu.VMEM_SHARED((N_PAD, DH), jnp.float32),
            pltpu.SemaphoreType.DMA,
            pltpu.SemaphoreType.DMA,
            pltpu.SemaphoreType.DMA,
        ],
    )
    def sc_agg(xh_hbm, col_hbm, row_hbm, a_hbm, out_hbm,
               col_v, row_v, a_v, rows0_v, rows1_v, rows2_v, rows3_v, agg_sh,
               sem_i, sem_g, sem_s):
        cid = lax.axis_index("c")
        sid = lax.axis_index("s")
        zero16 = jnp.zeros((16,), jnp.float32)
        rows_bufs = (rows0_v, rows1_v, rows2_v, rows3_v)

        def issue_idx(c):
            slot = lax.rem(c, NRING)
            rbase = sid * ERW + c * CR
            pltpu.async_copy(col_hbm.at[cid, pl.ds(rbase, CR)],
                             col_v.at[slot], sem_i)
            pltpu.async_copy(row_hbm.at[pl.ds(rbase, CR)],
                             row_v.at[slot], sem_i)
            pltpu.async_copy(a_hbm.at[pl.ds(sid * EW + c * CHUNK, CHUNK)],
                             a_v.at[slot], sem_i)

        def wait_idx(c):
            slot = lax.rem(c, NRING)
            rbase = sid * ERW + c * CR
            pltpu.make_async_copy(col_hbm.at[cid, pl.ds(rbase, CR)],
                                  col_v.at[slot], sem_i).wait()
            pltpu.make_async_copy(row_hbm.at[pl.ds(rbase, CR)],
                                  row_v.at[slot], sem_i).wait()
            pltpu.make_async_copy(a_hbm.at[pl.ds(sid * EW + c * CHUNK, CHUNK)],
                                  a_v.at[slot], sem_i).wait()

        def issue_gathers(c, buf):
            slot = lax.rem(c, NRING)
            for j in range(CR):
                pltpu.async_copy(xh_hbm.at[col_v.at[slot, j]],
                                 buf.at[pl.ds(j * 128, 128)], sem_g)

        def wait_gathers(c, buf):
            slot = lax.rem(c, NRING)
            for j in range(CR):
                pltpu.make_async_copy(xh_hbm.at[col_v.at[slot, j]],
                                      buf.at[pl.ds(j * 128, 128)],
                                      sem_g).wait()

        def issue_scatters(c, buf):
            slot = lax.rem(c, NRING)
            for j in range(CR):
                pltpu.async_copy(buf.at[pl.ds(j * 128, 128)],
                                 agg_sh.at[row_v.at[slot, j]],
                                 sem_s, add=True)

        def wait_scatters(c, buf):
            slot = lax.rem(c, NRING)
            for j in range(CR):
                pltpu.make_async_copy(buf.at[pl.ds(j * 128, 128)],
                                      agg_sh.at[row_v.at[slot, j]],
                                      sem_s).wait()

        issue_idx(0)
        issue_idx(1)

        def zrow(i, carry):
            for j in range(DH // 16):
                rows0_v[i, pl.ds(j * 16, 16)] = zero16
            return carry

        lax.fori_loop(0, ZROWS, zrow, 0)
        tile_row0 = sid * TROWS
        for k in range(TROWS // ZROWS):
            pltpu.sync_copy(rows0_v.at[pl.ds(0, ZROWS)],
                            agg_sh.at[pl.ds(tile_row0 + k * ZROWS, ZROWS)])
        wait_idx(0)
        issue_gathers(0, rows0_v)
        plsc.subcore_barrier()

        _dnums = lax.GatherDimensionNumbers(
            offset_dims=(), collapsed_slice_dims=(0,), start_index_map=(0,))

        def scale(c, buf):
            slot = lax.rem(c, NRING)
            nj = DH // 16

            def grp(g, carry2):
                a16 = a_v[slot, pl.ds(g * 16, 16)]
                for l0 in range(0, 16, 4):
                    es = [g * 16 + l0 + i for i in range(4)]
                    bcs = [
                        lax.gather(
                            a16, jnp.full((16, 1), l0 + i, jnp.int32),
                            _dnums, (1,),
                            mode=lax.GatherScatterMode.PROMISE_IN_BOUNDS)
                        for i in range(4)
                    ]
                    vals = [[buf[es[i], pl.ds(j * 16, 16)] for j in range(nj)]
                            for i in range(4)]
                    for i in range(4):
                        for j in range(nj):
                            buf[es[i], pl.ds(j * 16, 16)] = vals[i][j] * bcs[i]
                return carry2

            lax.fori_loop(0, CHUNK // 16, grp, 0)

        def body4(c4, carry):
            for b in range(NBUFS):
                c = c4 * NBUFS + b
                buf = rows_bufs[b]
                nbuf = rows_bufs[(b + 1) % NBUFS]
                wait_gathers(c, buf)

                @pl.when(c + 1 < NCHUNK)
                def _():
                    @pl.when(c >= NBUFS - 1)
                    def _():
                        wait_scatters(c - (NBUFS - 1), nbuf)

                    wait_idx(c + 1)
                    issue_gathers(c + 1, nbuf)

                @pl.when(c + 2 < NCHUNK)
                def _():
                    issue_idx(c + 2)

                scale(c, buf)
                issue_scatters(c, buf)
            return carry

        lax.fori_loop(0, NCHUNK // NBUFS, body4, 0)
        for c in range(NCHUNK - NBUFS, NCHUNK):
            wait_scatters(c, rows_bufs[c % NBUFS])

        plsc.subcore_barrier()
        for k in range(TROWS // ZROWS):
            r0 = tile_row0 + k * ZROWS
            pltpu.sync_copy(agg_sh.at[pl.ds(r0, ZROWS)],
                            rows0_v.at[pl.ds(0, ZROWS)])
            pltpu.sync_copy(rows0_v.at[pl.ds(0, ZROWS)],
                            out_hbm.at[cid, pl.ds(r0, ZROWS)])

    return sc_agg


_sc_agg = _build_sc_agg()

_MB = 1000


def _tc_body(agg_ref, w_ref, b_ref, g_ref, bt_ref, o_ref):
    acc = jnp.concatenate([agg_ref[0], agg_ref[1]], axis=-1)
    h = jnp.dot(acc, w_ref[...], preferred_element_type=jnp.float32)
    h = jnp.maximum(h + b_ref[...], 0.0)
    o_ref[...] = (g_ref[...] * h) / jnp.sqrt(jnp.float32(1.0 + 1e-3)) + bt_ref[...]


def _tc_finish(partials, W, b, gamma, beta):
    return pl.pallas_call(
        _tc_body,
        grid=(N_NODES // _MB,),
        in_specs=[
            pl.BlockSpec((2, _MB, DH), lambda i: (0, i, 0)),
            pl.BlockSpec((D, D), lambda i: (0, 0)),
            pl.BlockSpec((1, D), lambda i: (0, 0)),
            pl.BlockSpec((1, D), lambda i: (0, 0)),
            pl.BlockSpec((1, D), lambda i: (0, 0)),
        ],
        out_specs=pl.BlockSpec((_MB, D), lambda i: (i, 0)),
        out_shape=jax.ShapeDtypeStruct((N_NODES, D), jnp.float32),
    )(partials, W, b.reshape(1, D), gamma.reshape(1, D), beta.reshape(1, D))


def kernel(x, edge_index, a_values, W, b, gamma, beta):
    row = edge_index[0].astype(jnp.int32)
    col = edge_index[1].astype(jnp.int32)
    n_edges = row.shape[0]
    pad = E_PAD - n_edges
    ipad = jnp.zeros((pad,), jnp.int32)
    rowp = jnp.concatenate([row, ipad]).reshape(E_PAD // 128, 128)
    colp = jnp.concatenate([col, ipad]).reshape(E_PAD // 128, 128)
    colboth = jnp.stack([colp, colp + N_NODES])
    ap = jnp.concatenate([a_values, jnp.zeros((pad,), jnp.float32)])
    xh = jnp.concatenate([x[:, :DH], x[:, DH:]], axis=0)
    partials = _sc_agg(xh, colboth, rowp, ap)
    return _tc_finish(partials, W, b, gamma, beta)
